# baseline (speedup 1.0000x reference)
import functools

import jax
import jax.numpy as jnp
from jax import lax
from jax.experimental import pallas as pl
from jax.experimental.pallas import tpu as pltpu
from jax.experimental.pallas import tpu_sc as plsc

B, L, V, D = 4096, 200, 100000, 64

NC, NS = 2, 16
NW = NC * NS
BPW = B // NW
NG = BPW // 16

COL_BLK = 20480


def _score_body(embt_ref, w_ref, b_ref, out_ref):
    e = embt_ref[...]
    w = w_ref[...].T
    out_ref[...] = jnp.sum(e * w, axis=0) + b_ref[0]


def _scores(embt, wt, b):
    grid = (V + COL_BLK - 1) // COL_BLK
    return pl.pallas_call(
        _score_body,
        grid=(grid,),
        in_specs=[
            pl.BlockSpec((D, COL_BLK), lambda i: (0, i)),
            pl.BlockSpec((1, D), lambda i: (0, 0)),
            pl.BlockSpec(memory_space=pltpu.SMEM),
        ],
        out_specs=pl.BlockSpec((COL_BLK,), lambda i: (i,)),
        out_shape=jax.ShapeDtypeStruct((V,), jnp.float32),
    )(embt, wt, b)


def _pool_body(s_hbm, xt_hbm, out_hbm, s_v, x_v, o_v, sem_s, sem_x):
    wid = lax.axis_index("s") * NC + lax.axis_index("c")
    base = wid * BPW

    cp_s = pltpu.make_async_copy(s_hbm, s_v, sem_s)
    cp_s.start()
    cp_x = pltpu.make_async_copy(xt_hbm.at[:, :, pl.ds(base, BPW)], x_v, sem_x)
    cp_x.start()
    cp_s.wait()
    cp_x.wait()

    zero = jnp.zeros((16,), jnp.float32)

    @plsc.parallel_loop(0, L // 8, carry=(zero,) * NG)
    def accs(lt, accs):
        accs = list(accs)
        for ls in range(8):
            idxs = [x_v[lt, ls, pl.ds(g * 16, 16)] for g in range(NG)]
            vals = [plsc.load_gather(s_v, [idx]) for idx in idxs]
            for g in range(NG):
                accs[g] = accs[g] + vals[g]
        return tuple(accs)

    inv_l = jnp.float32(1.0 / L)
    for g in range(NG):
        z = accs[g] * inv_l
        o_v[pl.ds(g * 16, 16)] = 1.0 / (1.0 + jnp.exp(-z))

    pltpu.sync_copy(o_v, out_hbm.at[pl.ds(base, BPW)])


_pool = functools.partial(
    pl.kernel,
    out_type=jax.ShapeDtypeStruct((B,), jnp.float32),
    mesh=plsc.VectorSubcoreMesh(core_axis_name="c", subcore_axis_name="s"),
    compiler_params=pltpu.CompilerParams(needs_layout_passes=False),
    scratch_types=[
        pltpu.VMEM((V,), jnp.float32),
        pltpu.VMEM((L // 8, 8, BPW), jnp.int32),
        pltpu.VMEM((BPW,), jnp.float32),
        pltpu.SemaphoreType.DMA,
        pltpu.SemaphoreType.DMA,
    ],
)(_pool_body)


@jax.jit
def kernel(x, emb, W, b):
    s = _scores(emb.T, W, b)
    xt3 = x.astype(jnp.int32).T.reshape(L // 8, 8, B)
    return _pool(s, xt3)


# speedup vs baseline: 1.0297x; 1.0297x over previous
import functools

import jax
import jax.numpy as jnp
from jax import lax
from jax.experimental import pallas as pl
from jax.experimental.pallas import tpu as pltpu
from jax.experimental.pallas import tpu_sc as plsc

B, L, V, D = 4096, 200, 100000, 64

NC, NS = 2, 16
NW = NC * NS
BPW = B // NW
NG = BPW // 16

COL_BLK = 25600


def _score_body(embt_ref, w_ref, b_ref, out_ref):
    e = embt_ref[...]
    w = w_ref[...]
    s = jax.lax.dot_general(w, e, (((1,), (0,)), ((), ())),
                            preferred_element_type=jnp.float32)
    out_ref[...] = s[0] + b_ref[0]


def _scores(embt, wt, b):
    grid = (V + COL_BLK - 1) // COL_BLK
    return pl.pallas_call(
        _score_body,
        grid=(grid,),
        in_specs=[
            pl.BlockSpec((D, COL_BLK), lambda i: (0, i)),
            pl.BlockSpec((1, D), lambda i: (0, 0)),
            pl.BlockSpec(memory_space=pltpu.SMEM),
        ],
        out_specs=pl.BlockSpec((COL_BLK,), lambda i: (i,)),
        out_shape=jax.ShapeDtypeStruct((V,), jnp.float32),
    )(embt, wt, b)


def _pool_body(s_hbm, xt_hbm, out_hbm, s_v, x_v, o_v, sem_s, sem_x):
    wid = lax.axis_index("s") * NC + lax.axis_index("c")
    base = wid * BPW

    cp_s = pltpu.make_async_copy(s_hbm, s_v, sem_s)
    cp_s.start()
    cp_x = pltpu.make_async_copy(xt_hbm.at[:, :, pl.ds(base, BPW)], x_v, sem_x)
    cp_x.start()
    cp_s.wait()
    cp_x.wait()

    zero = jnp.zeros((16,), jnp.float32)

    @plsc.parallel_loop(0, L // 8, carry=(zero,) * NG)
    def accs(lt, accs):
        accs = list(accs)
        for ls in range(8):
            idxs = [x_v[lt, ls, pl.ds(g * 16, 16)] for g in range(NG)]
            vals = [plsc.load_gather(s_v, [idx]) for idx in idxs]
            for g in range(NG):
                accs[g] = accs[g] + vals[g]
        return tuple(accs)

    inv_l = jnp.float32(1.0 / L)
    for g in range(NG):
        z = accs[g] * inv_l
        o_v[pl.ds(g * 16, 16)] = 1.0 / (1.0 + jnp.exp(-z))

    pltpu.sync_copy(o_v, out_hbm.at[pl.ds(base, BPW)])


_pool = functools.partial(
    pl.kernel,
    out_type=jax.ShapeDtypeStruct((B,), jnp.float32),
    mesh=plsc.VectorSubcoreMesh(core_axis_name="c", subcore_axis_name="s"),
    compiler_params=pltpu.CompilerParams(needs_layout_passes=False),
    scratch_types=[
        pltpu.VMEM((V,), jnp.float32),
        pltpu.VMEM((L // 8, 8, BPW), jnp.int32),
        pltpu.VMEM((BPW,), jnp.float32),
        pltpu.SemaphoreType.DMA,
        pltpu.SemaphoreType.DMA,
    ],
)(_pool_body)


@jax.jit
def kernel(x, emb, W, b):
    s = _scores(emb.T, W, b)
    xt3 = x.astype(jnp.int32).T.reshape(L // 8, 8, B)
    return _pool(s, xt3)
